# baseline (speedup 1.0000x reference)
import numpy as np
import jax
import jax.numpy as jnp
from jax import lax
from jax.experimental import pallas as pl
from jax.experimental.pallas import tpu as pltpu
from jax.experimental.pallas import tpu_sc as plsc

VOCAB = 30522
D = 768
MAX_LEN = 512
NSEG = 3
B = 64
S = 512

NC = 2
NS = 16
NW = NC * NS
L = 16
SPW = S // NW
CHUNKS = D // L
NSLOT = 4


def _positional_encoding_np(max_len, d):
    position = np.arange(max_len, dtype=np.float32)[:, None]
    div_term = np.exp(np.arange(0, d, 2, dtype=np.float32) * -(np.log(10000.0) / d))
    pe = np.zeros((max_len, d), dtype=np.float32)
    pe[:, 0::2] = np.sin(position * div_term)
    pe[:, 1::2] = np.cos(position * div_term)
    return pe


_PE = _positional_encoding_np(MAX_LEN, D)


def _bert_embed_body(x_hbm, lbl_hbm, tok_hbm, seg_hbm, pe_hbm, out_hbm,
                     idx_v, lbl_v, pe_v, seg_v, posseg_v, rows_v,
                     *sems):
    g_sems = sems[:NSLOT]
    o_sems = sems[NSLOT:NSLOT * 2]

    wid = lax.axis_index("s") * NC + lax.axis_index("c")
    s0 = wid * SPW

    pltpu.sync_copy(x_hbm.at[pl.ds(wid * (B * SPW), B * SPW)], idx_v)
    pltpu.sync_copy(lbl_hbm.at[pl.ds(wid * (B * SPW), B * SPW)], lbl_v)
    pltpu.sync_copy(pe_hbm.at[pl.ds(s0, SPW), :], pe_v)
    pltpu.sync_copy(seg_hbm, seg_v)

    for lbl in range(NSEG):
        def _pp(j, _, lbl=lbl):
            row = lbl * SPW + j
            for k in range(CHUNKS):
                sl = pl.ds(k * L, L)
                posseg_v[row, sl] = pe_v[j, sl] + seg_v[lbl, sl]
            return 0
        lax.fori_loop(0, SPW, _pp, 0)

    def start_gather(b, slot):
        pltpu.async_copy(tok_hbm.at[idx_v.at[pl.ds(b * SPW, SPW)]],
                         rows_v.at[slot], g_sems[slot])

    def wait_gather(b, slot):
        pltpu.make_async_copy(tok_hbm.at[idx_v.at[pl.ds(b * SPW, SPW)]],
                              rows_v.at[slot], g_sems[slot]).wait()

    def start_out(b, slot):
        pltpu.async_copy(rows_v.at[slot], out_hbm.at[b, pl.ds(s0, SPW), :],
                         o_sems[slot])

    def wait_out(b, slot):
        pltpu.make_async_copy(rows_v.at[slot], out_hbm.at[b, pl.ds(s0, SPW), :],
                              o_sems[slot]).wait()

    KUNROLL = 4

    def compute(b, slot):
        lblvec = lbl_v[pl.ds(b * SPW, SPW)]
        for j in range(SPW):
            lbl = lblvec[j]
            row = lbl * SPW + j

            def _kk(k, _, j=j, row=row):
                for u in range(KUNROLL):
                    sl = pl.ds(k * (KUNROLL * L) + u * L, L)
                    rows_v[slot, j, sl] = rows_v[slot, j, sl] + posseg_v[row, sl]
                return 0
            lax.fori_loop(0, CHUNKS // KUNROLL, _kk, 0)

    def pipe(b, _):
        start_gather(b, 0)
        wait_gather(b, 0)
        compute(b, 0)
        start_out(b, 0)
        wait_out(b, 0)
        return 0

    lax.fori_loop(0, B, pipe, 0)


@jax.jit
def _bert_embed(x, segment_label, token_table, segment_table, pe):
    mesh = plsc.VectorSubcoreMesh(core_axis_name="c", subcore_axis_name="s",
                                  num_cores=NC, num_subcores=NS)
    scratch = [
        pltpu.VMEM((B * SPW,), jnp.int32),
        pltpu.VMEM((B * SPW,), jnp.int32),
        pltpu.VMEM((SPW, D), jnp.float32),
        pltpu.VMEM((8, D), jnp.float32),
        pltpu.VMEM((NSEG * SPW, D), jnp.float32),
        pltpu.VMEM((NSLOT, SPW, D), jnp.float32),
    ] + [pltpu.SemaphoreType.DMA] * (2 * NSLOT)
    f = pl.kernel(
        _bert_embed_body,
        out_type=jax.ShapeDtypeStruct((B, S, D), jnp.float32),
        mesh=mesh,
        scratch_types=scratch,
    )
    def _worker_major(a):
        return a.reshape(B, NW, SPW).transpose(1, 0, 2).reshape(NW * B * SPW)

    seg_pad = jnp.zeros((8, D), jnp.float32).at[:NSEG].set(segment_table)
    return f(_worker_major(x), _worker_major(segment_label),
             token_table, seg_pad, pe)


def kernel(x, segment_label, token_table, segment_table):
    pe = jnp.asarray(_PE)
    return _bert_embed(x, segment_label, token_table, segment_table, pe)


# speedup vs baseline: 1.3166x; 1.3166x over previous
import numpy as np
import jax
import jax.numpy as jnp
from jax import lax
from jax.experimental import pallas as pl
from jax.experimental.pallas import tpu as pltpu
from jax.experimental.pallas import tpu_sc as plsc

VOCAB = 30522
D = 768
MAX_LEN = 512
NSEG = 3
B = 64
S = 512

NC = 2
NS = 16
NW = NC * NS
L = 16
SPW = S // NW
CHUNKS = D // L
NSLOT = 4


def _positional_encoding_np(max_len, d):
    position = np.arange(max_len, dtype=np.float32)[:, None]
    div_term = np.exp(np.arange(0, d, 2, dtype=np.float32) * -(np.log(10000.0) / d))
    pe = np.zeros((max_len, d), dtype=np.float32)
    pe[:, 0::2] = np.sin(position * div_term)
    pe[:, 1::2] = np.cos(position * div_term)
    return pe


_PE = _positional_encoding_np(MAX_LEN, D)


def _bert_embed_body(x_hbm, lbl_hbm, tok_hbm, seg_hbm, pe_hbm, out_hbm,
                     idx_v, lbl_v, pe_v, seg_v, posseg_v, rows_v,
                     *sems):
    g_sems = sems[:NSLOT]
    o_sems = sems[NSLOT:NSLOT * 2]

    wid = lax.axis_index("s") * NC + lax.axis_index("c")
    s0 = wid * SPW

    pltpu.sync_copy(x_hbm.at[pl.ds(wid * (B * SPW), B * SPW)], idx_v)
    pltpu.sync_copy(lbl_hbm.at[pl.ds(wid * (B * SPW), B * SPW)], lbl_v)
    pltpu.sync_copy(pe_hbm.at[pl.ds(s0, SPW), :], pe_v)
    pltpu.sync_copy(seg_hbm, seg_v)

    for lbl in range(NSEG):
        def _pp(j, _, lbl=lbl):
            row = lbl * SPW + j
            for k in range(CHUNKS):
                sl = pl.ds(k * L, L)
                posseg_v[row, sl] = pe_v[j, sl] + seg_v[lbl, sl]
            return 0
        lax.fori_loop(0, SPW, _pp, 0)

    def start_gather(b, slot):
        pltpu.async_copy(tok_hbm.at[idx_v.at[pl.ds(b * SPW, SPW)]],
                         rows_v.at[slot], g_sems[slot])

    def wait_gather(b, slot):
        pltpu.make_async_copy(tok_hbm.at[idx_v.at[pl.ds(b * SPW, SPW)]],
                              rows_v.at[slot], g_sems[slot]).wait()

    def start_out(b, slot):
        pltpu.async_copy(rows_v.at[slot], out_hbm.at[b, pl.ds(s0, SPW), :],
                         o_sems[slot])

    def wait_out(b, slot):
        pltpu.make_async_copy(rows_v.at[slot], out_hbm.at[b, pl.ds(s0, SPW), :],
                              o_sems[slot]).wait()

    KUNROLL = 4

    def compute(b, slot):
        lblvec = lbl_v[pl.ds(b * SPW, SPW)]
        for j in range(SPW):
            lbl = lblvec[j]
            row = lbl * SPW + j

            def _kk(k, _, j=j, row=row):
                for u in range(KUNROLL):
                    sl = pl.ds(k * (KUNROLL * L) + u * L, L)
                    rows_v[slot, j, sl] = rows_v[slot, j, sl] + posseg_v[row, sl]
                return 0
            lax.fori_loop(0, CHUNKS // KUNROLL, _kk, 0)

    def step(b, slot, do_wait_out, do_gather):
        nslot = (slot + 2) % NSLOT
        if do_wait_out:
            wait_out(b - 2, nslot)
        if do_gather:
            start_gather(b + 2, nslot)
        wait_gather(b, slot)
        compute(b, slot)
        start_out(b, slot)

    start_gather(0, 0)
    start_gather(1, 1)
    step(0, 0, False, True)
    step(1, 1, False, True)

    def pipe(t, _):
        b = NSLOT * t + 2
        for i in range(NSLOT):
            step(b + i, (2 + i) % NSLOT, True, True)
        return 0

    lax.fori_loop(0, (B - 4) // NSLOT, pipe, 0)

    step(B - 2, (B - 2) % NSLOT, True, False)
    step(B - 1, (B - 1) % NSLOT, True, False)
    wait_out(B - 2, (B - 2) % NSLOT)
    wait_out(B - 1, (B - 1) % NSLOT)


@jax.jit
def _bert_embed(x, segment_label, token_table, segment_table, pe):
    mesh = plsc.VectorSubcoreMesh(core_axis_name="c", subcore_axis_name="s",
                                  num_cores=NC, num_subcores=NS)
    scratch = [
        pltpu.VMEM((B * SPW,), jnp.int32),
        pltpu.VMEM((B * SPW,), jnp.int32),
        pltpu.VMEM((SPW, D), jnp.float32),
        pltpu.VMEM((8, D), jnp.float32),
        pltpu.VMEM((NSEG * SPW, D), jnp.float32),
        pltpu.VMEM((NSLOT, SPW, D), jnp.float32),
    ] + [pltpu.SemaphoreType.DMA] * (2 * NSLOT)
    f = pl.kernel(
        _bert_embed_body,
        out_type=jax.ShapeDtypeStruct((B, S, D), jnp.float32),
        mesh=mesh,
        scratch_types=scratch,
    )
    def _worker_major(a):
        return a.reshape(B, NW, SPW).transpose(1, 0, 2).reshape(NW * B * SPW)

    seg_pad = jnp.zeros((8, D), jnp.float32).at[:NSEG].set(segment_table)
    return f(_worker_major(x), _worker_major(segment_label),
             token_table, seg_pad, pe)


def kernel(x, segment_label, token_table, segment_table):
    pe = jnp.asarray(_PE)
    return _bert_embed(x, segment_label, token_table, segment_table, pe)


# speedup vs baseline: 1.4332x; 1.0886x over previous
import numpy as np
import jax
import jax.numpy as jnp
from jax import lax
from jax.experimental import pallas as pl
from jax.experimental.pallas import tpu as pltpu
from jax.experimental.pallas import tpu_sc as plsc

VOCAB = 30522
D = 768
MAX_LEN = 512
NSEG = 3
B = 64
S = 512

NC = 2
NS = 16
NW = NC * NS
L = 16
SPW = S // NW
CHUNKS = D // L
NSLOT = 4


def _positional_encoding_np(max_len, d):
    position = np.arange(max_len, dtype=np.float32)[:, None]
    div_term = np.exp(np.arange(0, d, 2, dtype=np.float32) * -(np.log(10000.0) / d))
    pe = np.zeros((max_len, d), dtype=np.float32)
    pe[:, 0::2] = np.sin(position * div_term)
    pe[:, 1::2] = np.cos(position * div_term)
    return pe


_PE = _positional_encoding_np(MAX_LEN, D)


def _bert_embed_body(x_hbm, lbl_hbm, tok_hbm, seg_hbm, pe_hbm, out_hbm,
                     idx_v, lbl_v, pe_v, seg_v, posseg_v, rows_v,
                     *sems):
    g_sems = sems[:NSLOT]
    o_sems = sems[NSLOT:NSLOT * 2]

    wid = lax.axis_index("s") * NC + lax.axis_index("c")
    s0 = wid * SPW

    pltpu.sync_copy(x_hbm.at[pl.ds(wid * (B * SPW), B * SPW)], idx_v)
    pltpu.sync_copy(lbl_hbm.at[pl.ds(wid * (B * SPW), B * SPW)],
                    lbl_v.at[pl.ds(0, B * SPW)])
    pltpu.sync_copy(pe_hbm.at[pl.ds(s0, SPW), :], pe_v)
    pltpu.sync_copy(seg_hbm, seg_v)

    for lbl in range(NSEG):
        def _pp(j, _, lbl=lbl):
            row = lbl * SPW + j
            for k in range(CHUNKS):
                sl = pl.ds(k * L, L)
                posseg_v[row, sl] = pe_v[j, sl] + seg_v[lbl, sl]
            return 0
        lax.fori_loop(0, SPW, _pp, 0)

    def start_gather(b, slot):
        pltpu.async_copy(tok_hbm.at[idx_v.at[pl.ds(b * SPW, SPW)]],
                         rows_v.at[slot], g_sems[slot])

    def wait_gather(b, slot):
        pltpu.make_async_copy(tok_hbm.at[idx_v.at[pl.ds(b * SPW, SPW)]],
                              rows_v.at[slot], g_sems[slot]).wait()

    def start_out(b, slot):
        pltpu.async_copy(rows_v.at[slot], out_hbm.at[b, pl.ds(s0, SPW), :],
                         o_sems[slot])

    def wait_out(b, slot):
        pltpu.make_async_copy(rows_v.at[slot], out_hbm.at[b, pl.ds(s0, SPW), :],
                              o_sems[slot]).wait()

    def compute(b, slot):
        def _j(j, _):
            lbl = lbl_v[pl.ds(b * SPW + j, L)][0]
            row = lbl * SPW + j
            for k in range(CHUNKS):
                sl = pl.ds(k * L, L)
                rows_v[slot, j, sl] = rows_v[slot, j, sl] + posseg_v[row, sl]
            return 0
        lax.fori_loop(0, SPW, _j, 0)

    def step(b, slot, do_wait_out, do_gather):
        nslot = (slot + 2) % NSLOT
        if do_wait_out:
            wait_out(b - 2, nslot)
        if do_gather:
            start_gather(b + 2, nslot)
        wait_gather(b, slot)
        compute(b, slot)
        start_out(b, slot)

    start_gather(0, 0)
    start_gather(1, 1)
    step(0, 0, False, True)
    step(1, 1, False, True)

    def pipe(t, _):
        b = NSLOT * t + 2
        for i in range(NSLOT):
            step(b + i, (2 + i) % NSLOT, True, True)
        return 0

    lax.fori_loop(0, (B - 4) // NSLOT, pipe, 0)

    step(B - 2, (B - 2) % NSLOT, True, False)
    step(B - 1, (B - 1) % NSLOT, True, False)
    wait_out(B - 2, (B - 2) % NSLOT)
    wait_out(B - 1, (B - 1) % NSLOT)


@jax.jit
def _bert_embed(x, segment_label, token_table, segment_table, pe):
    mesh = plsc.VectorSubcoreMesh(core_axis_name="c", subcore_axis_name="s",
                                  num_cores=NC, num_subcores=NS)
    scratch = [
        pltpu.VMEM((B * SPW,), jnp.int32),
        pltpu.VMEM((B * SPW + L,), jnp.int32),
        pltpu.VMEM((SPW, D), jnp.float32),
        pltpu.VMEM((8, D), jnp.float32),
        pltpu.VMEM((NSEG * SPW, D), jnp.float32),
        pltpu.VMEM((NSLOT, SPW, D), jnp.float32),
    ] + [pltpu.SemaphoreType.DMA] * (2 * NSLOT)
    f = pl.kernel(
        _bert_embed_body,
        out_type=jax.ShapeDtypeStruct((B, S, D), jnp.float32),
        mesh=mesh,
        scratch_types=scratch,
    )
    def _worker_major(a):
        return a.reshape(B, NW, SPW).transpose(1, 0, 2).reshape(NW * B * SPW)

    seg_pad = jnp.zeros((8, D), jnp.float32).at[:NSEG].set(segment_table)
    return f(_worker_major(x), _worker_major(segment_label),
             token_table, seg_pad, pe)


def kernel(x, segment_label, token_table, segment_table):
    pe = jnp.asarray(_PE)
    return _bert_embed(x, segment_label, token_table, segment_table, pe)


# speedup vs baseline: 3.3885x; 2.3642x over previous
import numpy as np
import jax
import jax.numpy as jnp
from jax import lax
from jax.experimental import pallas as pl
from jax.experimental.pallas import tpu as pltpu
from jax.experimental.pallas import tpu_sc as plsc

VOCAB = 30522
D = 768
MAX_LEN = 512
NSEG = 3
B = 64
S = 512

NC = 2
NS = 16
NW = NC * NS
L = 16
SPW = S // NW
CHUNKS = D // L
NSLOT = 4


def _positional_encoding_np(max_len, d):
    position = np.arange(max_len, dtype=np.float32)[:, None]
    div_term = np.exp(np.arange(0, d, 2, dtype=np.float32) * -(np.log(10000.0) / d))
    pe = np.zeros((max_len, d), dtype=np.float32)
    pe[:, 0::2] = np.sin(position * div_term)
    pe[:, 1::2] = np.cos(position * div_term)
    return pe


_PE = _positional_encoding_np(MAX_LEN, D)


def _bert_embed_body(x_hbm, lbl_hbm, tok_hbm, seg_hbm, pe_hbm, out_hbm,
                     idx_v, lbl_v, pe_v, seg_v, posseg_v, rows_v,
                     *sems):
    g_sems = sems[:NSLOT]
    o_sems = sems[NSLOT:NSLOT * 2]

    wid = lax.axis_index("s") * NC + lax.axis_index("c")
    s0 = wid * SPW

    pltpu.sync_copy(x_hbm.at[pl.ds(wid * (B * SPW), B * SPW)], idx_v)
    pltpu.sync_copy(lbl_hbm.at[pl.ds(wid * (B * SPW), B * SPW)],
                    lbl_v.at[pl.ds(0, B * SPW)])
    pltpu.sync_copy(pe_hbm.at[pl.ds(s0, SPW), :], pe_v)
    pltpu.sync_copy(seg_hbm, seg_v)

    for lbl in range(NSEG):
        def _pp(j, _, lbl=lbl):
            row = lbl * SPW + j
            for k in range(CHUNKS):
                sl = pl.ds(k * L, L)
                posseg_v[row, sl] = pe_v[j, sl] + seg_v[lbl, sl]
            return 0
        lax.fori_loop(0, SPW, _pp, 0)

    def start_gather(b, slot):
        pltpu.async_copy(tok_hbm.at[idx_v.at[pl.ds(b * SPW, SPW)]],
                         rows_v.at[slot], g_sems[slot])

    def wait_gather(b, slot):
        pltpu.make_async_copy(tok_hbm.at[idx_v.at[pl.ds(b * SPW, SPW)]],
                              rows_v.at[slot], g_sems[slot]).wait()

    def start_out(b, slot):
        pltpu.async_copy(rows_v.at[slot], out_hbm.at[b, pl.ds(s0, SPW), :],
                         o_sems[slot])

    def wait_out(b, slot):
        pltpu.make_async_copy(rows_v.at[slot], out_hbm.at[b, pl.ds(s0, SPW), :],
                              o_sems[slot]).wait()

    def compute(b, slot):
        def _j(j, _):
            lbl = lbl_v[pl.ds(b * SPW + j, L)][0]
            row = lbl * SPW + j

            @plsc.parallel_loop(0, D, L, unroll=8)
            def _k(c):
                sl = pl.ds(c, L)
                rows_v[slot, j, sl] = rows_v[slot, j, sl] + posseg_v[row, sl]
            return 0
        lax.fori_loop(0, SPW, _j, 0)

    def step(b, slot, do_wait_out, do_gather):
        nslot = (slot + 2) % NSLOT
        if do_wait_out:
            wait_out(b - 2, nslot)
        if do_gather:
            start_gather(b + 2, nslot)
        wait_gather(b, slot)
        compute(b, slot)
        start_out(b, slot)

    start_gather(0, 0)
    start_gather(1, 1)
    step(0, 0, False, True)
    step(1, 1, False, True)

    def pipe(t, _):
        b = NSLOT * t + 2
        for i in range(NSLOT):
            step(b + i, (2 + i) % NSLOT, True, True)
        return 0

    lax.fori_loop(0, (B - 4) // NSLOT, pipe, 0)

    step(B - 2, (B - 2) % NSLOT, True, False)
    step(B - 1, (B - 1) % NSLOT, True, False)
    wait_out(B - 2, (B - 2) % NSLOT)
    wait_out(B - 1, (B - 1) % NSLOT)


@jax.jit
def _bert_embed(x, segment_label, token_table, segment_table, pe):
    mesh = plsc.VectorSubcoreMesh(core_axis_name="c", subcore_axis_name="s",
                                  num_cores=NC, num_subcores=NS)
    scratch = [
        pltpu.VMEM((B * SPW,), jnp.int32),
        pltpu.VMEM((B * SPW + L,), jnp.int32),
        pltpu.VMEM((SPW, D), jnp.float32),
        pltpu.VMEM((8, D), jnp.float32),
        pltpu.VMEM((NSEG * SPW, D), jnp.float32),
        pltpu.VMEM((NSLOT, SPW, D), jnp.float32),
    ] + [pltpu.SemaphoreType.DMA] * (2 * NSLOT)
    f = pl.kernel(
        _bert_embed_body,
        out_type=jax.ShapeDtypeStruct((B, S, D), jnp.float32),
        mesh=mesh,
        scratch_types=scratch,
    )
    def _worker_major(a):
        return a.reshape(B, NW, SPW).transpose(1, 0, 2).reshape(NW * B * SPW)

    seg_pad = jnp.zeros((8, D), jnp.float32).at[:NSEG].set(segment_table)
    return f(_worker_major(x), _worker_major(segment_label),
             token_table, seg_pad, pe)


def kernel(x, segment_label, token_table, segment_table):
    pe = jnp.asarray(_PE)
    return _bert_embed(x, segment_label, token_table, segment_table, pe)


# speedup vs baseline: 3.6498x; 1.0771x over previous
import numpy as np
import jax
import jax.numpy as jnp
from jax import lax
from jax.experimental import pallas as pl
from jax.experimental.pallas import tpu as pltpu
from jax.experimental.pallas import tpu_sc as plsc

VOCAB = 30522
D = 768
MAX_LEN = 512
NSEG = 3
B = 64
S = 512

NC = 2
NS = 16
NW = NC * NS
L = 16
SPW = S // NW
CHUNKS = D // L
NSLOT = 4


def _positional_encoding_np(max_len, d):
    position = np.arange(max_len, dtype=np.float32)[:, None]
    div_term = np.exp(np.arange(0, d, 2, dtype=np.float32) * -(np.log(10000.0) / d))
    pe = np.zeros((max_len, d), dtype=np.float32)
    pe[:, 0::2] = np.sin(position * div_term)
    pe[:, 1::2] = np.cos(position * div_term)
    return pe


_PE = _positional_encoding_np(MAX_LEN, D)


def _bert_embed_body(x_hbm, lbl_hbm, tok_hbm, seg_hbm, pe_hbm, out_hbm,
                     idx_v, lbl_v, pe_v, seg_v, posseg_v, rows_v,
                     *sems):
    g_sems = sems[:NSLOT]
    o_sems = sems[NSLOT:NSLOT * 2]

    wid = lax.axis_index("s") * NC + lax.axis_index("c")
    s0 = wid * SPW

    pltpu.sync_copy(x_hbm.at[pl.ds(wid * (B * SPW), B * SPW)], idx_v)
    pltpu.sync_copy(lbl_hbm.at[pl.ds(wid * (B * SPW), B * SPW)],
                    lbl_v.at[pl.ds(0, B * SPW)])
    pltpu.sync_copy(pe_hbm.at[pl.ds(s0, SPW), :], pe_v)
    pltpu.sync_copy(seg_hbm, seg_v)

    for lbl in range(NSEG):
        def _pp(j, _, lbl=lbl):
            row = lbl * SPW + j

            @plsc.parallel_loop(0, D, L, unroll=8)
            def _pk(c):
                sl = pl.ds(c, L)
                posseg_v[row, sl] = pe_v[j, sl] + seg_v[lbl, sl]
            return 0
        lax.fori_loop(0, SPW, _pp, 0)

    def start_gather(b, slot):
        pltpu.async_copy(tok_hbm.at[idx_v.at[pl.ds(b * SPW, SPW)]],
                         rows_v.at[slot], g_sems[slot])

    def wait_gather(b, slot):
        pltpu.make_async_copy(tok_hbm.at[idx_v.at[pl.ds(b * SPW, SPW)]],
                              rows_v.at[slot], g_sems[slot]).wait()

    def start_out(b, slot):
        pltpu.async_copy(rows_v.at[slot], out_hbm.at[b, pl.ds(s0, SPW), :],
                         o_sems[slot])

    def wait_out(b, slot):
        pltpu.make_async_copy(rows_v.at[slot], out_hbm.at[b, pl.ds(s0, SPW), :],
                              o_sems[slot]).wait()

    def compute(b, slot):
        def _j(j, _):
            lbl = lbl_v[pl.ds(b * SPW + j, L)][0]
            row = lbl * SPW + j

            @plsc.parallel_loop(0, D, L, unroll=8)
            def _k(c):
                sl = pl.ds(c, L)
                rows_v[slot, j, sl] = rows_v[slot, j, sl] + posseg_v[row, sl]
            return 0
        lax.fori_loop(0, SPW, _j, 0)

    def step(b, slot, do_wait_out, do_gather):
        nslot = (slot + 2) % NSLOT
        if do_wait_out:
            wait_out(b - 2, nslot)
        if do_gather:
            start_gather(b + 2, nslot)
        wait_gather(b, slot)
        compute(b, slot)
        start_out(b, slot)

    start_gather(0, 0)
    start_gather(1, 1)
    step(0, 0, False, True)
    step(1, 1, False, True)

    def pipe(t, _):
        b = NSLOT * t + 2
        for i in range(NSLOT):
            step(b + i, (2 + i) % NSLOT, True, True)
        return 0

    lax.fori_loop(0, (B - 4) // NSLOT, pipe, 0)

    step(B - 2, (B - 2) % NSLOT, True, False)
    step(B - 1, (B - 1) % NSLOT, True, False)
    wait_out(B - 2, (B - 2) % NSLOT)
    wait_out(B - 1, (B - 1) % NSLOT)


@jax.jit
def _bert_embed(x, segment_label, token_table, segment_table, pe):
    mesh = plsc.VectorSubcoreMesh(core_axis_name="c", subcore_axis_name="s",
                                  num_cores=NC, num_subcores=NS)
    scratch = [
        pltpu.VMEM((B * SPW,), jnp.int32),
        pltpu.VMEM((B * SPW + L,), jnp.int32),
        pltpu.VMEM((SPW, D), jnp.float32),
        pltpu.VMEM((8, D), jnp.float32),
        pltpu.VMEM((NSEG * SPW, D), jnp.float32),
        pltpu.VMEM((NSLOT, SPW, D), jnp.float32),
    ] + [pltpu.SemaphoreType.DMA] * (2 * NSLOT)
    f = pl.kernel(
        _bert_embed_body,
        out_type=jax.ShapeDtypeStruct((B, S, D), jnp.float32),
        mesh=mesh,
        scratch_types=scratch,
    )
    def _worker_major(a):
        return a.reshape(B, NW, SPW).transpose(1, 0, 2).reshape(NW * B * SPW)

    seg_pad = jnp.zeros((8, D), jnp.float32).at[:NSEG].set(segment_table)
    return f(_worker_major(x), _worker_major(segment_label),
             token_table, seg_pad, pe)


def kernel(x, segment_label, token_table, segment_table):
    pe = jnp.asarray(_PE)
    return _bert_embed(x, segment_label, token_table, segment_table, pe)
